# Initial kernel scaffold; baseline (speedup 1.0000x reference)
import jax
import jax.numpy as jnp
from jax.experimental import pallas as pl


def kernel(inputs, Wpre, bpre, Wpost, bpost, codebook):
    raise NotImplementedError("write your pallas kernel here")


# speedup vs baseline: 3.1506x; 3.1506x over previous
import functools

import jax
import jax.numpy as jnp
from jax import lax
from jax.experimental import pallas as pl
from jax.experimental.pallas import tpu as pltpu
from jax.experimental.pallas import tpu_sc as plsc

B, L, D_IN = 16, 256, 384
D = 4048
K = 512
N = B * L
BLK = 512
NSTEPS = N // BLK

_DOT11 = (((1,), (1,)), ((), ()))


def _main_body(in_ref, wpre_ref, bpre_ref, cb_ref, idx_ref, loss_ref, csum_scr):
    @pl.when(pl.program_id(0) == 0)
    def _init():
        sq = cb_ref[...] * cb_ref[...]
        ones = jnp.ones((1, D), jnp.float32)
        csum_scr[...] = lax.dot_general(
            ones, sq, _DOT11, preferred_element_type=jnp.float32)
        loss_ref[...] = jnp.zeros_like(loss_ref)

    x = lax.dot_general(in_ref[...], wpre_ref[...], _DOT11,
                        preferred_element_type=jnp.float32) + bpre_ref[...]
    a = jnp.sum(x * x, axis=1, keepdims=True)
    dm = lax.dot_general(x, cb_ref[...], _DOT11,
                         preferred_element_type=jnp.float32)
    dist = (a - 2.0 * dm) + csum_scr[...]
    neg = -dist
    m = jnp.max(neg, axis=1, keepdims=True)
    ids = lax.broadcasted_iota(jnp.int32, dist.shape, 1)
    idx_ref[...] = jnp.min(jnp.where(neg == m, ids, K), axis=1, keepdims=True)
    loss_ref[...] = loss_ref[...] + jnp.sum(-m)

    @pl.when(pl.program_id(0) == NSTEPS - 1)
    def _fin():
        loss_ref[...] = loss_ref[...] * (1.25 / (N * D))


def _prep_body(cb_ref, wpost_ref, bpost_ref, wq_ref):
    wq_ref[...] = lax.dot_general(cb_ref[...], wpost_ref[...], _DOT11,
                                  preferred_element_type=jnp.float32) + bpost_ref[...]


_main_call = pl.pallas_call(
    _main_body,
    grid=(NSTEPS,),
    in_specs=[
        pl.BlockSpec((BLK, D_IN), lambda i: (i, 0)),
        pl.BlockSpec((D, D_IN), lambda i: (0, 0)),
        pl.BlockSpec((1, D), lambda i: (0, 0)),
        pl.BlockSpec((K, D), lambda i: (0, 0)),
    ],
    out_specs=[
        pl.BlockSpec((BLK, 1), lambda i: (i, 0)),
        pl.BlockSpec((1, 1), lambda i: (0, 0)),
    ],
    out_shape=[
        jax.ShapeDtypeStruct((N, 1), jnp.int32),
        jax.ShapeDtypeStruct((1, 1), jnp.float32),
    ],
    scratch_shapes=[pltpu.VMEM((1, K), jnp.float32)],
)

_prep_call = pl.pallas_call(
    _prep_body,
    in_specs=[
        pl.BlockSpec((K, D), lambda: (0, 0)),
        pl.BlockSpec((D_IN, D), lambda: (0, 0)),
        pl.BlockSpec((1, D_IN), lambda: (0, 0)),
    ],
    out_specs=pl.BlockSpec((K, D_IN), lambda: (0, 0)),
    out_shape=jax.ShapeDtypeStruct((K, D_IN), jnp.float32),
)

_NC, _NS = 2, 16
_NW = _NC * _NS
_B_PER_W = N // _NW


@functools.partial(
    pl.kernel,
    mesh=plsc.VectorSubcoreMesh(core_axis_name="c", subcore_axis_name="s"),
    out_type=jax.ShapeDtypeStruct((N, D_IN), jnp.float32),
    scratch_types=[
        pltpu.VMEM((_B_PER_W,), jnp.int32),
        pltpu.VMEM((_B_PER_W, D_IN), jnp.float32),
        pltpu.SemaphoreType.DMA,
    ],
)
def _gather_call(table_hbm, idx_hbm, out_hbm, idx_v, rows_v, sem):
    wid = lax.axis_index("s") * _NC + lax.axis_index("c")
    base = wid * _B_PER_W
    pltpu.sync_copy(idx_hbm.at[pl.ds(base, _B_PER_W)], idx_v)
    pltpu.async_copy(table_hbm.at[idx_v], rows_v, sem).wait()
    pltpu.sync_copy(rows_v, out_hbm.at[pl.ds(base, _B_PER_W)])


def kernel(inputs, Wpre, bpre, Wpost, bpost, codebook):
    flat_in = inputs.reshape(N, D_IN)
    idx2d, loss = _main_call(flat_in, Wpre, bpre.reshape(1, D), codebook)
    wq = _prep_call(codebook, Wpost, bpost.reshape(1, D_IN))
    out = _gather_call(wq, idx2d.reshape(N))
    return out.reshape(B, L, D_IN), loss.reshape(())


# speedup vs baseline: 3.1882x; 1.0119x over previous
import functools

import jax
import jax.numpy as jnp
from jax import lax
from jax.experimental import pallas as pl
from jax.experimental.pallas import tpu as pltpu
from jax.experimental.pallas import tpu_sc as plsc

B, L, D_IN = 16, 256, 384
D = 4048
K = 512
N = B * L
BLK = 512
NSTEPS = N // BLK

_DOT11 = (((1,), (1,)), ((), ()))


def _main_body(in_ref, wpre_ref, bpre_ref, cb_ref, wpost_ref, bpost_ref,
               idx_ref, loss_ref, wq_ref, csum_scr):
    @pl.when(pl.program_id(0) == 0)
    def _init():
        sq = cb_ref[...] * cb_ref[...]
        ones = jnp.ones((1, D), jnp.float32)
        csum_scr[...] = lax.dot_general(
            ones, sq, _DOT11, preferred_element_type=jnp.float32)
        loss_ref[...] = jnp.zeros_like(loss_ref)

    x = lax.dot_general(in_ref[...], wpre_ref[...], _DOT11,
                        preferred_element_type=jnp.float32) + bpre_ref[...]
    a = jnp.sum(x * x, axis=1, keepdims=True)
    dm = lax.dot_general(x, cb_ref[...], _DOT11,
                         preferred_element_type=jnp.float32)
    dist = (a - 2.0 * dm) + csum_scr[...]
    neg = -dist
    m = jnp.max(neg, axis=1, keepdims=True)
    ids = lax.broadcasted_iota(jnp.int32, dist.shape, 1)
    idx_ref[...] = jnp.min(jnp.where(neg == m, ids, K), axis=1, keepdims=True)
    loss_ref[...] = loss_ref[...] + jnp.sum(-m)

    @pl.when(pl.program_id(0) == NSTEPS - 1)
    def _fin():
        loss_ref[...] = loss_ref[...] * (1.25 / (N * D))
        wq_ref[...] = lax.dot_general(cb_ref[...], wpost_ref[...], _DOT11,
                                      preferred_element_type=jnp.float32) + bpost_ref[...]


_main_call = pl.pallas_call(
    _main_body,
    grid=(NSTEPS,),
    in_specs=[
        pl.BlockSpec((BLK, D_IN), lambda i: (i, 0)),
        pl.BlockSpec((D, D_IN), lambda i: (0, 0)),
        pl.BlockSpec((1, D), lambda i: (0, 0)),
        pl.BlockSpec((K, D), lambda i: (0, 0)),
        pl.BlockSpec((D_IN, D), lambda i: (0, 0)),
        pl.BlockSpec((1, D_IN), lambda i: (0, 0)),
    ],
    out_specs=[
        pl.BlockSpec((BLK, 1), lambda i: (i, 0)),
        pl.BlockSpec((1, 1), lambda i: (0, 0)),
        pl.BlockSpec((K, D_IN), lambda i: (0, 0)),
    ],
    out_shape=[
        jax.ShapeDtypeStruct((N, 1), jnp.int32),
        jax.ShapeDtypeStruct((1, 1), jnp.float32),
        jax.ShapeDtypeStruct((K, D_IN), jnp.float32),
    ],
    scratch_shapes=[pltpu.VMEM((1, K), jnp.float32)],
)

_NC, _NS = 2, 16
_NW = _NC * _NS
_B_PER_W = N // _NW


@functools.partial(
    pl.kernel,
    mesh=plsc.VectorSubcoreMesh(core_axis_name="c", subcore_axis_name="s"),
    out_type=jax.ShapeDtypeStruct((N, D_IN), jnp.float32),
    scratch_types=[
        pltpu.VMEM((_B_PER_W,), jnp.int32),
        pltpu.VMEM((_B_PER_W, D_IN), jnp.float32),
        pltpu.SemaphoreType.DMA,
    ],
)
def _gather_call(table_hbm, idx_hbm, out_hbm, idx_v, rows_v, sem):
    wid = lax.axis_index("s") * _NC + lax.axis_index("c")
    base = wid * _B_PER_W
    pltpu.sync_copy(idx_hbm.at[pl.ds(base, _B_PER_W)], idx_v)
    pltpu.async_copy(table_hbm.at[idx_v], rows_v, sem).wait()
    pltpu.sync_copy(rows_v, out_hbm.at[pl.ds(base, _B_PER_W)])


def kernel(inputs, Wpre, bpre, Wpost, bpost, codebook):
    flat_in = inputs.reshape(N, D_IN)
    idx2d, loss, wq = _main_call(flat_in, Wpre, bpre.reshape(1, D), codebook,
                                 Wpost, bpost.reshape(1, D_IN))
    out = _gather_call(wq, idx2d.reshape(N))
    return out.reshape(B, L, D_IN), loss.reshape(())


# speedup vs baseline: 3.2410x; 1.0166x over previous
import functools

import jax
import jax.numpy as jnp
from jax import lax
from jax.experimental import pallas as pl
from jax.experimental.pallas import tpu as pltpu
from jax.experimental.pallas import tpu_sc as plsc

B, L, D_IN = 16, 256, 384
D = 4048
K = 512
N = B * L
BLK = 1024
NSTEPS = N // BLK

_DOT11 = (((1,), (1,)), ((), ()))


def _main_body(in_ref, wpre_ref, bpre_ref, cb_ref, wpost_ref, bpost_ref,
               idx_ref, loss_ref, wq_ref, csum_scr):
    @pl.when(pl.program_id(0) == 0)
    def _init():
        sq = cb_ref[...] * cb_ref[...]
        ones = jnp.ones((1, D), jnp.float32)
        csum_scr[...] = lax.dot_general(
            ones, sq, _DOT11, preferred_element_type=jnp.float32)
        loss_ref[...] = jnp.zeros_like(loss_ref)

    x = lax.dot_general(in_ref[...], wpre_ref[...], _DOT11,
                        preferred_element_type=jnp.float32) + bpre_ref[...]
    a = jnp.sum(x * x, axis=1, keepdims=True)
    dm = lax.dot_general(x, cb_ref[...], _DOT11,
                         preferred_element_type=jnp.float32)
    dist = (a - 2.0 * dm) + csum_scr[...]
    neg = -dist
    m = jnp.max(neg, axis=1, keepdims=True)
    ids = lax.broadcasted_iota(jnp.int32, dist.shape, 1)
    idx_ref[...] = jnp.min(jnp.where(neg == m, ids, K), axis=1, keepdims=True)
    loss_ref[...] = loss_ref[...] + jnp.sum(-m)

    @pl.when(pl.program_id(0) == NSTEPS - 1)
    def _fin():
        loss_ref[...] = loss_ref[...] * (1.25 / (N * D))
        wq_ref[...] = lax.dot_general(cb_ref[...], wpost_ref[...], _DOT11,
                                      preferred_element_type=jnp.float32) + bpost_ref[...]


_main_call = pl.pallas_call(
    _main_body,
    grid=(NSTEPS,),
    in_specs=[
        pl.BlockSpec((BLK, D_IN), lambda i: (i, 0)),
        pl.BlockSpec((D, D_IN), lambda i: (0, 0)),
        pl.BlockSpec((1, D), lambda i: (0, 0)),
        pl.BlockSpec((K, D), lambda i: (0, 0)),
        pl.BlockSpec((D_IN, D), lambda i: (0, 0)),
        pl.BlockSpec((1, D_IN), lambda i: (0, 0)),
    ],
    out_specs=[
        pl.BlockSpec((BLK, 1), lambda i: (i, 0)),
        pl.BlockSpec((1, 1), lambda i: (0, 0)),
        pl.BlockSpec((K, D_IN), lambda i: (0, 0)),
    ],
    out_shape=[
        jax.ShapeDtypeStruct((N, 1), jnp.int32),
        jax.ShapeDtypeStruct((1, 1), jnp.float32),
        jax.ShapeDtypeStruct((K, D_IN), jnp.float32),
    ],
    scratch_shapes=[pltpu.VMEM((1, K), jnp.float32)],
)

_NC, _NS = 2, 16
_NW = _NC * _NS
_B_PER_W = N // _NW


@functools.partial(
    pl.kernel,
    mesh=plsc.VectorSubcoreMesh(core_axis_name="c", subcore_axis_name="s"),
    out_type=jax.ShapeDtypeStruct((N, D_IN), jnp.float32),
    scratch_types=[
        pltpu.VMEM((_B_PER_W,), jnp.int32),
        pltpu.VMEM((_B_PER_W, D_IN), jnp.float32),
        pltpu.SemaphoreType.DMA,
    ],
)
def _gather_call(table_hbm, idx_hbm, out_hbm, idx_v, rows_v, sem):
    wid = lax.axis_index("s") * _NC + lax.axis_index("c")
    base = wid * _B_PER_W
    pltpu.sync_copy(idx_hbm.at[pl.ds(base, _B_PER_W)], idx_v)
    pltpu.async_copy(table_hbm.at[idx_v], rows_v, sem).wait()
    pltpu.sync_copy(rows_v, out_hbm.at[pl.ds(base, _B_PER_W)])


def kernel(inputs, Wpre, bpre, Wpost, bpost, codebook):
    flat_in = inputs.reshape(N, D_IN)
    idx2d, loss, wq = _main_call(flat_in, Wpre, bpre.reshape(1, D), codebook,
                                 Wpost, bpost.reshape(1, D_IN))
    out = _gather_call(wq, idx2d.reshape(N))
    return out.reshape(B, L, D_IN), loss.reshape(())
